# baseline (speedup 1.0000x reference)
import dataclasses
import functools

import jax
import jax.numpy as jnp
from jax import lax
from jax.experimental import pallas as pl
from jax.experimental.pallas import tpu as pltpu
from jax.experimental.pallas import tpu_sc as plsc

Q = 1024
D = 64
K = 100000
KB = 2000
NB = K // KB
SB = 8
IBIG = 2**30
FBIG = 3.0e38


def _nn_body(x2t_ref, xsq_ref, keys_ref, ksq_ref, out_ref, rmin_ref, ridx_ref):
    i = pl.program_id(0)
    kb = keys_ref[...]
    ksq = ksq_ref[...]
    m = lax.dot_general(
        kb, x2t_ref[...],
        dimension_numbers=(((1,), (0,)), ((), ())),
        preferred_element_type=jnp.float32,
    )

    @pl.when(i == 0)
    def _():
        rmin_ref[...] = jnp.full((SB, Q), FBIG, jnp.float32)
        ridx_ref[...] = jnp.zeros((SB, Q), jnp.int32)

    acc = rmin_ref[...]
    aidx = ridx_ref[...]
    xsq = xsq_ref[...]
    for s in range(KB // SB):
        sl = slice(s * SB, (s + 1) * SB)
        d = (ksq[sl] + xsq) - m[sl]
        mask = d < acc
        acc = jnp.where(mask, d, acc)
        aidx = jnp.where(mask, i * KB + s * SB, aidx)
    rmin_ref[...] = acc
    ridx_ref[...] = aidx

    @pl.when(i == NB - 1)
    def _():
        gmin = jnp.min(acc, axis=0, keepdims=True)
        gidx = aidx + lax.broadcasted_iota(jnp.int32, (SB, Q), 0)
        cand = jnp.where(acc == gmin, gidx, IBIG)
        out_ref[...] = jnp.min(cand, axis=0, keepdims=True)


def _nn_argmin(x2t, xsq, keys, ksqc, interpret=False):
    return pl.pallas_call(
        _nn_body,
        grid=(NB,),
        in_specs=[
            pl.BlockSpec((D, Q), lambda i: (0, 0)),
            pl.BlockSpec((1, Q), lambda i: (0, 0)),
            pl.BlockSpec((KB, D), lambda i: (i, 0)),
            pl.BlockSpec((KB, 1), lambda i: (i, 0)),
        ],
        out_specs=pl.BlockSpec((1, Q), lambda i: (0, 0)),
        out_shape=jax.ShapeDtypeStruct((1, Q), jnp.int32),
        scratch_shapes=[
            pltpu.VMEM((SB, Q), jnp.float32),
            pltpu.VMEM((SB, Q), jnp.int32),
        ],
        interpret=interpret,
    )(x2t, xsq, keys, ksqc)


def _sc_compiler_params():
    cp = pltpu.CompilerParams()
    if "needs_layout_passes" in pltpu.CompilerParams.__dataclass_fields__:
        cp = dataclasses.replace(cp, needs_layout_passes=False)
    return cp


def _label_gather(labels, nn_idx):
    mesh = plsc.VectorSubcoreMesh(core_axis_name="c", subcore_axis_name="s")

    @functools.partial(
        pl.kernel,
        mesh=mesh,
        out_type=jax.ShapeDtypeStruct((Q,), labels.dtype),
        scratch_types=[
            pltpu.VMEM((K,), labels.dtype),
            pltpu.VMEM((Q,), jnp.int32),
            pltpu.VMEM((Q,), labels.dtype),
        ],
        compiler_params=_sc_compiler_params(),
    )
    def gather_kernel(labels_hbm, idx_hbm, out_hbm, lab_v, idx_v, out_v):
        cid = lax.axis_index("c")
        sid = lax.axis_index("s")

        @pl.when(jnp.logical_and(cid == 0, sid == 0))
        def _():
            pltpu.sync_copy(labels_hbm, lab_v)
            pltpu.sync_copy(idx_hbm, idx_v)
            for j in range(Q // 16):
                ids = idx_v[pl.ds(j * 16, 16)]
                out_v[pl.ds(j * 16, 16)] = plsc.load_gather(lab_v, [ids])
            pltpu.sync_copy(out_v, out_hbm)

    return gather_kernel(labels, nn_idx)


def kernel(x, keys, labels):
    xsq = jnp.sum(x * x, axis=1, keepdims=True)
    ksqc = jnp.sum(keys * keys, axis=1, keepdims=True)
    nn_idx = _nn_argmin((x + x).T, xsq.T, keys, ksqc)
    return _label_gather(labels, nn_idx.reshape(Q))


# speedup vs baseline: 1.0386x; 1.0386x over previous
import dataclasses
import functools

import jax
import jax.numpy as jnp
from jax import lax
from jax.experimental import pallas as pl
from jax.experimental.pallas import tpu as pltpu
from jax.experimental.pallas import tpu_sc as plsc

Q = 1024
D = 64
K = 100000
KB = 2000
NB = K // KB
SB = 8
IBIG = 2**30
FBIG = 3.0e38


def _nn_body(x2t_ref, xsq_ref, keys_ref, out_ref, rmin_ref, ridx_ref):
    i = pl.program_id(0)
    kb = keys_ref[...]
    ksq = jnp.sum(kb * kb, axis=1, keepdims=True)
    m = lax.dot_general(
        kb, x2t_ref[...],
        dimension_numbers=(((1,), (0,)), ((), ())),
        preferred_element_type=jnp.float32,
    )

    @pl.when(i == 0)
    def _():
        rmin_ref[...] = jnp.full((SB, Q), FBIG, jnp.float32)
        ridx_ref[...] = jnp.zeros((SB, Q), jnp.int32)

    acc = rmin_ref[...]
    aidx = ridx_ref[...]
    xsq = xsq_ref[...]
    for s in range(KB // SB):
        sl = slice(s * SB, (s + 1) * SB)
        d = (ksq[sl] + xsq) - m[sl]
        mask = d < acc
        acc = jnp.where(mask, d, acc)
        aidx = jnp.where(mask, i * KB + s * SB, aidx)
    rmin_ref[...] = acc
    ridx_ref[...] = aidx

    @pl.when(i == NB - 1)
    def _():
        gmin = jnp.min(acc, axis=0, keepdims=True)
        gidx = aidx + lax.broadcasted_iota(jnp.int32, (SB, Q), 0)
        cand = jnp.where(acc == gmin, gidx, IBIG)
        out_ref[...] = jnp.min(cand, axis=0, keepdims=True)


def _nn_argmin(x2t, xsq, keys, interpret=False):
    return pl.pallas_call(
        _nn_body,
        grid=(NB,),
        in_specs=[
            pl.BlockSpec((D, Q), lambda i: (0, 0)),
            pl.BlockSpec((1, Q), lambda i: (0, 0)),
            pl.BlockSpec((KB, D), lambda i: (i, 0)),
        ],
        out_specs=pl.BlockSpec((1, Q), lambda i: (0, 0)),
        out_shape=jax.ShapeDtypeStruct((1, Q), jnp.int32),
        scratch_shapes=[
            pltpu.VMEM((SB, Q), jnp.float32),
            pltpu.VMEM((SB, Q), jnp.int32),
        ],
        interpret=interpret,
    )(x2t, xsq, keys)


def _sc_compiler_params():
    cp = pltpu.CompilerParams()
    if "needs_layout_passes" in pltpu.CompilerParams.__dataclass_fields__:
        cp = dataclasses.replace(cp, needs_layout_passes=False)
    return cp


def _label_gather(labels, nn_idx):
    mesh = plsc.VectorSubcoreMesh(core_axis_name="c", subcore_axis_name="s")

    @functools.partial(
        pl.kernel,
        mesh=mesh,
        out_type=jax.ShapeDtypeStruct((Q,), labels.dtype),
        scratch_types=[
            pltpu.VMEM((K,), labels.dtype),
            pltpu.VMEM((Q,), jnp.int32),
            pltpu.VMEM((Q,), labels.dtype),
        ],
        compiler_params=_sc_compiler_params(),
    )
    def gather_kernel(labels_hbm, idx_hbm, out_hbm, lab_v, idx_v, out_v):
        cid = lax.axis_index("c")
        sid = lax.axis_index("s")

        @pl.when(jnp.logical_and(cid == 0, sid == 0))
        def _():
            pltpu.sync_copy(labels_hbm, lab_v)
            pltpu.sync_copy(idx_hbm, idx_v)
            for j in range(Q // 16):
                ids = idx_v[pl.ds(j * 16, 16)]
                out_v[pl.ds(j * 16, 16)] = plsc.load_gather(lab_v, [ids])
            pltpu.sync_copy(out_v, out_hbm)

    return gather_kernel(labels, nn_idx)


def kernel(x, keys, labels):
    xsq = jnp.sum(x * x, axis=1, keepdims=True)
    nn_idx = _nn_argmin((x + x).T, xsq.T, keys)
    return _label_gather(labels, nn_idx.reshape(Q))


# speedup vs baseline: 1.2499x; 1.2035x over previous
import dataclasses
import functools

import jax
import jax.numpy as jnp
from jax import lax
from jax.experimental import pallas as pl
from jax.experimental.pallas import tpu as pltpu
from jax.experimental.pallas import tpu_sc as plsc

Q = 1024
D = 64
K = 100000
KB = 2048
NB = pl.cdiv(K, KB)
KTAIL = K - (NB - 1) * KB
SB = 8
IBIG = 2**30
FBIG = 3.0e38


def _nn_body(x2t_ref, xsq_ref, keyst_ref, out_ref, rmin_ref, ridx_ref):
    i = pl.program_id(0)
    kb = keyst_ref[...].T
    ksq = jnp.sum(kb * kb, axis=1, keepdims=True)
    m = lax.dot_general(
        kb, x2t_ref[...],
        dimension_numbers=(((1,), (0,)), ((), ())),
        preferred_element_type=jnp.float32,
    )
    xsq = xsq_ref[...]

    @pl.when(i == 0)
    def _():
        rmin_ref[...] = jnp.full((SB, Q), FBIG, jnp.float32)
        ridx_ref[...] = jnp.zeros((SB, Q), jnp.int32)

    def scan_strips(n_strips):
        acc = rmin_ref[...]
        aidx = ridx_ref[...]
        for s in range(n_strips):
            sl = slice(s * SB, (s + 1) * SB)
            d = (ksq[sl] + xsq) - m[sl]
            mask = d < acc
            acc = jnp.where(mask, d, acc)
            aidx = jnp.where(mask, i * KB + s * SB, aidx)
        rmin_ref[...] = acc
        ridx_ref[...] = aidx
        return acc, aidx

    @pl.when(i < NB - 1)
    def _():
        scan_strips(KB // SB)

    @pl.when(i == NB - 1)
    def _():
        acc, aidx = scan_strips(KTAIL // SB)
        gmin = jnp.min(acc, axis=0, keepdims=True)
        gidx = aidx + lax.broadcasted_iota(jnp.int32, (SB, Q), 0)
        cand = jnp.where(acc == gmin, gidx, IBIG)
        out_ref[...] = jnp.min(cand, axis=0, keepdims=True)


def _nn_argmin(x2t, xsq, keyst, interpret=False):
    return pl.pallas_call(
        _nn_body,
        grid=(NB,),
        in_specs=[
            pl.BlockSpec((D, Q), lambda i: (0, 0)),
            pl.BlockSpec((1, Q), lambda i: (0, 0)),
            pl.BlockSpec((D, KB), lambda i: (0, i)),
        ],
        out_specs=pl.BlockSpec((1, Q), lambda i: (0, 0)),
        out_shape=jax.ShapeDtypeStruct((1, Q), jnp.int32),
        scratch_shapes=[
            pltpu.VMEM((SB, Q), jnp.float32),
            pltpu.VMEM((SB, Q), jnp.int32),
        ],
        interpret=interpret,
    )(x2t, xsq, keyst)


def _sc_compiler_params():
    cp = pltpu.CompilerParams()
    if "needs_layout_passes" in pltpu.CompilerParams.__dataclass_fields__:
        cp = dataclasses.replace(cp, needs_layout_passes=False)
    return cp


def _label_gather(labels, nn_idx):
    mesh = plsc.VectorSubcoreMesh(core_axis_name="c", subcore_axis_name="s")

    @functools.partial(
        pl.kernel,
        mesh=mesh,
        out_type=jax.ShapeDtypeStruct((Q,), labels.dtype),
        scratch_types=[
            pltpu.VMEM((K,), labels.dtype),
            pltpu.VMEM((Q,), jnp.int32),
            pltpu.VMEM((Q,), labels.dtype),
        ],
        compiler_params=_sc_compiler_params(),
    )
    def gather_kernel(labels_hbm, idx_hbm, out_hbm, lab_v, idx_v, out_v):
        cid = lax.axis_index("c")
        sid = lax.axis_index("s")

        @pl.when(jnp.logical_and(cid == 0, sid == 0))
        def _():
            pltpu.sync_copy(labels_hbm, lab_v)
            pltpu.sync_copy(idx_hbm, idx_v)
            for j in range(Q // 16):
                ids = idx_v[pl.ds(j * 16, 16)]
                out_v[pl.ds(j * 16, 16)] = plsc.load_gather(lab_v, [ids])
            pltpu.sync_copy(out_v, out_hbm)

    return gather_kernel(labels, nn_idx)


def kernel(x, keys, labels):
    xsq = jnp.sum(x * x, axis=1, keepdims=True)
    nn_idx = _nn_argmin((x + x).T, xsq.T, keys.T)
    return _label_gather(labels, nn_idx.reshape(Q))


# speedup vs baseline: 1.3200x; 1.0561x over previous
import dataclasses
import functools

import jax
import jax.numpy as jnp
from jax import lax
from jax.experimental import pallas as pl
from jax.experimental.pallas import tpu as pltpu
from jax.experimental.pallas import tpu_sc as plsc

Q = 1024
D = 64
K = 100000
KB = 2048
NB = pl.cdiv(K, KB)
KTAIL = K - (NB - 1) * KB
SB = 8
IBIG = 2**30
FBIG = 3.0e38


CH = 256
NCH = KB // CH
SPC = CH // SB


def _nn_body(x2t_ref, xsq_ref, keyst_ref, out_ref, rmin_ref, ridx_ref,
             m0_ref, k0_ref, m1_ref, k1_ref):
    i = pl.program_id(0)
    xsq = xsq_ref[...]

    @pl.when(i == 0)
    def _():
        rmin_ref[...] = jnp.full((SB, Q), FBIG, jnp.float32)
        ridx_ref[...] = jnp.zeros((SB, Q), jnp.int32)

    def compute_chunk(mw_ref, kw_ref, c):
        csl = slice(c * CH, (c + 1) * CH)
        kb = keyst_ref[:, csl].T
        kw_ref[csl] = jnp.sum(kb * kb, axis=1, keepdims=True)
        mw_ref[csl] = lax.dot_general(
            kb, x2t_ref[...],
            dimension_numbers=(((1,), (0,)), ((), ())),
            preferred_element_type=jnp.float32,
        )

    def step_mid(mw_ref, kw_ref, mr_ref, kr_ref):
        base = (i - 1) * KB
        acc = rmin_ref[...]
        aidx = ridx_ref[...]
        for c in range(NCH):
            compute_chunk(mw_ref, kw_ref, c)
            for s in range(c * SPC, (c + 1) * SPC):
                sl = slice(s * SB, (s + 1) * SB)
                d = (kr_ref[sl] + xsq) - mr_ref[sl]
                mask = d < acc
                acc = jnp.where(mask, d, acc)
                aidx = jnp.where(mask, base + s * SB, aidx)
        rmin_ref[...] = acc
        ridx_ref[...] = aidx

    @pl.when(i == 0)
    def _():
        for c in range(NCH):
            compute_chunk(m0_ref, k0_ref, c)

    @pl.when(jnp.logical_and(i > 0, jnp.logical_and(i < NB, i % 2 == 1)))
    def _():
        step_mid(m1_ref, k1_ref, m0_ref, k0_ref)

    @pl.when(jnp.logical_and(i > 0, jnp.logical_and(i < NB, i % 2 == 0)))
    def _():
        step_mid(m0_ref, k0_ref, m1_ref, k1_ref)

    @pl.when(i == NB)
    def _():
        mr_ref, kr_ref = (m0_ref, k0_ref) if (NB - 1) % 2 == 0 else (m1_ref, k1_ref)
        base = (NB - 1) * KB
        acc = rmin_ref[...]
        aidx = ridx_ref[...]
        for s in range(KTAIL // SB):
            sl = slice(s * SB, (s + 1) * SB)
            d = (kr_ref[sl] + xsq) - mr_ref[sl]
            mask = d < acc
            acc = jnp.where(mask, d, acc)
            aidx = jnp.where(mask, base + s * SB, aidx)
        gmin = jnp.min(acc, axis=0, keepdims=True)
        gidx = aidx + lax.broadcasted_iota(jnp.int32, (SB, Q), 0)
        cand = jnp.where(acc == gmin, gidx, IBIG)
        out_ref[...] = jnp.min(cand, axis=0, keepdims=True)


def _nn_argmin(x2t, xsq, keyst, interpret=False):
    return pl.pallas_call(
        _nn_body,
        grid=(NB + 1,),
        in_specs=[
            pl.BlockSpec((D, Q), lambda i: (0, 0)),
            pl.BlockSpec((1, Q), lambda i: (0, 0)),
            pl.BlockSpec((D, KB), lambda i: (0, jnp.minimum(i, NB - 1))),
        ],
        out_specs=pl.BlockSpec((1, Q), lambda i: (0, 0)),
        out_shape=jax.ShapeDtypeStruct((1, Q), jnp.int32),
        scratch_shapes=[
            pltpu.VMEM((SB, Q), jnp.float32),
            pltpu.VMEM((SB, Q), jnp.int32),
            pltpu.VMEM((KB, Q), jnp.float32),
            pltpu.VMEM((KB, 1), jnp.float32),
            pltpu.VMEM((KB, Q), jnp.float32),
            pltpu.VMEM((KB, 1), jnp.float32),
        ],
        interpret=interpret,
    )(x2t, xsq, keyst)


def _sc_compiler_params():
    cp = pltpu.CompilerParams()
    if "needs_layout_passes" in pltpu.CompilerParams.__dataclass_fields__:
        cp = dataclasses.replace(cp, needs_layout_passes=False)
    return cp


def _label_gather(labels, nn_idx):
    mesh = plsc.VectorSubcoreMesh(core_axis_name="c", subcore_axis_name="s")

    @functools.partial(
        pl.kernel,
        mesh=mesh,
        out_type=jax.ShapeDtypeStruct((Q,), labels.dtype),
        scratch_types=[
            pltpu.VMEM((K,), labels.dtype),
            pltpu.VMEM((Q,), jnp.int32),
            pltpu.VMEM((Q,), labels.dtype),
        ],
        compiler_params=_sc_compiler_params(),
    )
    def gather_kernel(labels_hbm, idx_hbm, out_hbm, lab_v, idx_v, out_v):
        cid = lax.axis_index("c")
        sid = lax.axis_index("s")

        @pl.when(jnp.logical_and(cid == 0, sid == 0))
        def _():
            pltpu.sync_copy(labels_hbm, lab_v)
            pltpu.sync_copy(idx_hbm, idx_v)
            for j in range(Q // 16):
                ids = idx_v[pl.ds(j * 16, 16)]
                out_v[pl.ds(j * 16, 16)] = plsc.load_gather(lab_v, [ids])
            pltpu.sync_copy(out_v, out_hbm)

    return gather_kernel(labels, nn_idx)


def kernel(x, keys, labels):
    xsq = jnp.sum(x * x, axis=1, keepdims=True)
    nn_idx = _nn_argmin((x + x).T, xsq.T, keys.T)
    return _label_gather(labels, nn_idx.reshape(Q))


# speedup vs baseline: 1.3663x; 1.0351x over previous
import dataclasses
import functools

import jax
import jax.numpy as jnp
from jax import lax
from jax.experimental import pallas as pl
from jax.experimental.pallas import tpu as pltpu
from jax.experimental.pallas import tpu_sc as plsc

Q = 1024
D = 64
K = 100000
KB = 2048
NB = pl.cdiv(K, KB)
KTAIL = K - (NB - 1) * KB
SB = 8
IBIG = 2**30
FBIG = 3.0e38


CH = 512
NCH = KB // CH
SPC = CH // SB


def _nn_body(x2t_ref, xsq_ref, keyst_ref, out_ref, rmin_ref, ridx_ref,
             m0_ref, k0_ref, m1_ref, k1_ref):
    i = pl.program_id(0)
    xsq = xsq_ref[...]

    @pl.when(i == 0)
    def _():
        rmin_ref[...] = jnp.full((SB, Q), FBIG, jnp.float32)
        ridx_ref[...] = jnp.zeros((SB, Q), jnp.int32)

    def compute_chunk(mw_ref, kw_ref, c):
        csl = slice(c * CH, (c + 1) * CH)
        kb = keyst_ref[:, csl].T
        kw_ref[csl] = jnp.sum(kb * kb, axis=1, keepdims=True)
        mw_ref[csl] = lax.dot_general(
            kb, x2t_ref[...],
            dimension_numbers=(((1,), (0,)), ((), ())),
            preferred_element_type=jnp.float32,
        )

    def step_mid(mw_ref, kw_ref, mr_ref, kr_ref):
        base = (i - 1) * KB
        acc = rmin_ref[...]
        aidx = ridx_ref[...]
        for c in range(NCH):
            compute_chunk(mw_ref, kw_ref, c)
            for s in range(c * SPC, (c + 1) * SPC):
                sl = slice(s * SB, (s + 1) * SB)
                d = (kr_ref[sl] + xsq) - mr_ref[sl]
                mask = d < acc
                acc = jnp.where(mask, d, acc)
                aidx = jnp.where(mask, base + s * SB, aidx)
        rmin_ref[...] = acc
        ridx_ref[...] = aidx

    @pl.when(i == 0)
    def _():
        for c in range(NCH):
            compute_chunk(m0_ref, k0_ref, c)

    @pl.when(jnp.logical_and(i > 0, jnp.logical_and(i < NB, i % 2 == 1)))
    def _():
        step_mid(m1_ref, k1_ref, m0_ref, k0_ref)

    @pl.when(jnp.logical_and(i > 0, jnp.logical_and(i < NB, i % 2 == 0)))
    def _():
        step_mid(m0_ref, k0_ref, m1_ref, k1_ref)

    @pl.when(i == NB)
    def _():
        mr_ref, kr_ref = (m0_ref, k0_ref) if (NB - 1) % 2 == 0 else (m1_ref, k1_ref)
        base = (NB - 1) * KB
        acc = rmin_ref[...]
        aidx = ridx_ref[...]
        for s in range(KTAIL // SB):
            sl = slice(s * SB, (s + 1) * SB)
            d = (kr_ref[sl] + xsq) - mr_ref[sl]
            mask = d < acc
            acc = jnp.where(mask, d, acc)
            aidx = jnp.where(mask, base + s * SB, aidx)
        gmin = jnp.min(acc, axis=0, keepdims=True)
        gidx = aidx + lax.broadcasted_iota(jnp.int32, (SB, Q), 0)
        cand = jnp.where(acc == gmin, gidx, IBIG)
        out_ref[...] = jnp.min(cand, axis=0, keepdims=True)


def _nn_argmin(x2t, xsq, keyst, interpret=False):
    return pl.pallas_call(
        _nn_body,
        grid=(NB + 1,),
        in_specs=[
            pl.BlockSpec((D, Q), lambda i: (0, 0)),
            pl.BlockSpec((1, Q), lambda i: (0, 0)),
            pl.BlockSpec((D, KB), lambda i: (0, jnp.minimum(i, NB - 1))),
        ],
        out_specs=pl.BlockSpec((1, Q), lambda i: (0, 0)),
        out_shape=jax.ShapeDtypeStruct((1, Q), jnp.int32),
        scratch_shapes=[
            pltpu.VMEM((SB, Q), jnp.float32),
            pltpu.VMEM((SB, Q), jnp.int32),
            pltpu.VMEM((KB, Q), jnp.float32),
            pltpu.VMEM((KB, 1), jnp.float32),
            pltpu.VMEM((KB, Q), jnp.float32),
            pltpu.VMEM((KB, 1), jnp.float32),
        ],
        interpret=interpret,
    )(x2t, xsq, keyst)


def _sc_compiler_params():
    cp = pltpu.CompilerParams()
    if "needs_layout_passes" in pltpu.CompilerParams.__dataclass_fields__:
        cp = dataclasses.replace(cp, needs_layout_passes=False)
    return cp


def _label_gather(labels, nn_idx):
    mesh = plsc.VectorSubcoreMesh(core_axis_name="c", subcore_axis_name="s")

    @functools.partial(
        pl.kernel,
        mesh=mesh,
        out_type=jax.ShapeDtypeStruct((Q,), labels.dtype),
        scratch_types=[
            pltpu.VMEM((K,), labels.dtype),
            pltpu.VMEM((Q,), jnp.int32),
            pltpu.VMEM((Q,), labels.dtype),
        ],
        compiler_params=_sc_compiler_params(),
    )
    def gather_kernel(labels_hbm, idx_hbm, out_hbm, lab_v, idx_v, out_v):
        cid = lax.axis_index("c")
        sid = lax.axis_index("s")

        @pl.when(jnp.logical_and(cid == 0, sid == 0))
        def _():
            pltpu.sync_copy(labels_hbm, lab_v)
            pltpu.sync_copy(idx_hbm, idx_v)
            for j in range(Q // 16):
                ids = idx_v[pl.ds(j * 16, 16)]
                out_v[pl.ds(j * 16, 16)] = plsc.load_gather(lab_v, [ids])
            pltpu.sync_copy(out_v, out_hbm)

    return gather_kernel(labels, nn_idx)


def kernel(x, keys, labels):
    xsq = jnp.sum(x * x, axis=1, keepdims=True)
    nn_idx = _nn_argmin((x + x).T, xsq.T, keys.T)
    return _label_gather(labels, nn_idx.reshape(Q))


# speedup vs baseline: 1.4335x; 1.0492x over previous
import dataclasses
import functools

import jax
import jax.numpy as jnp
from jax import lax
from jax.experimental import pallas as pl
from jax.experimental.pallas import tpu as pltpu
from jax.experimental.pallas import tpu_sc as plsc

Q = 1024
D = 64
K = 100000
KB = 2048
NB = pl.cdiv(K, KB)
KTAIL = K - (NB - 1) * KB
SB = 8
IBIG = 2**30
FBIG = 3.0e38


CH = 1024
NCH = KB // CH
SPC = CH // SB


def _nn_body(x2t_ref, xsq_ref, keyst_ref, out_ref, rmin_ref, ridx_ref,
             m0_ref, k0_ref, m1_ref, k1_ref):
    i = pl.program_id(0)
    xsq = xsq_ref[...]

    @pl.when(i == 0)
    def _():
        rmin_ref[...] = jnp.full((SB, Q), FBIG, jnp.float32)
        ridx_ref[...] = jnp.zeros((SB, Q), jnp.int32)

    def compute_chunk(mw_ref, kw_ref, c):
        csl = slice(c * CH, (c + 1) * CH)
        kb = keyst_ref[:, csl].T
        kw_ref[csl] = jnp.sum(kb * kb, axis=1, keepdims=True)
        mw_ref[csl] = lax.dot_general(
            kb, x2t_ref[...],
            dimension_numbers=(((1,), (0,)), ((), ())),
            preferred_element_type=jnp.float32,
        )

    def step_mid(mw_ref, kw_ref, mr_ref, kr_ref):
        base = (i - 1) * KB
        acc = rmin_ref[...]
        aidx = ridx_ref[...]
        for c in range(NCH):
            compute_chunk(mw_ref, kw_ref, c)
            for s in range(c * SPC, (c + 1) * SPC):
                sl = slice(s * SB, (s + 1) * SB)
                d = (kr_ref[sl] + xsq) - mr_ref[sl]
                mask = d < acc
                acc = jnp.where(mask, d, acc)
                aidx = jnp.where(mask, base + s * SB, aidx)
        rmin_ref[...] = acc
        ridx_ref[...] = aidx

    @pl.when(i == 0)
    def _():
        for c in range(NCH):
            compute_chunk(m0_ref, k0_ref, c)

    @pl.when(jnp.logical_and(i > 0, jnp.logical_and(i < NB, i % 2 == 1)))
    def _():
        step_mid(m1_ref, k1_ref, m0_ref, k0_ref)

    @pl.when(jnp.logical_and(i > 0, jnp.logical_and(i < NB, i % 2 == 0)))
    def _():
        step_mid(m0_ref, k0_ref, m1_ref, k1_ref)

    @pl.when(i == NB)
    def _():
        mr_ref, kr_ref = (m0_ref, k0_ref) if (NB - 1) % 2 == 0 else (m1_ref, k1_ref)
        base = (NB - 1) * KB
        acc = rmin_ref[...]
        aidx = ridx_ref[...]
        for s in range(KTAIL // SB):
            sl = slice(s * SB, (s + 1) * SB)
            d = (kr_ref[sl] + xsq) - mr_ref[sl]
            mask = d < acc
            acc = jnp.where(mask, d, acc)
            aidx = jnp.where(mask, base + s * SB, aidx)
        gmin = jnp.min(acc, axis=0, keepdims=True)
        gidx = aidx + lax.broadcasted_iota(jnp.int32, (SB, Q), 0)
        cand = jnp.where(acc == gmin, gidx, IBIG)
        out_ref[...] = jnp.min(cand, axis=0, keepdims=True)


def _nn_argmin(x2t, xsq, keyst, interpret=False):
    return pl.pallas_call(
        _nn_body,
        grid=(NB + 1,),
        in_specs=[
            pl.BlockSpec((D, Q), lambda i: (0, 0)),
            pl.BlockSpec((1, Q), lambda i: (0, 0)),
            pl.BlockSpec((D, KB), lambda i: (0, jnp.minimum(i, NB - 1))),
        ],
        out_specs=pl.BlockSpec((1, Q), lambda i: (0, 0)),
        out_shape=jax.ShapeDtypeStruct((1, Q), jnp.int32),
        scratch_shapes=[
            pltpu.VMEM((SB, Q), jnp.float32),
            pltpu.VMEM((SB, Q), jnp.int32),
            pltpu.VMEM((KB, Q), jnp.float32),
            pltpu.VMEM((KB, 1), jnp.float32),
            pltpu.VMEM((KB, Q), jnp.float32),
            pltpu.VMEM((KB, 1), jnp.float32),
        ],
        interpret=interpret,
    )(x2t, xsq, keyst)


def _sc_compiler_params():
    cp = pltpu.CompilerParams()
    if "needs_layout_passes" in pltpu.CompilerParams.__dataclass_fields__:
        cp = dataclasses.replace(cp, needs_layout_passes=False)
    return cp


def _label_gather(labels, nn_idx):
    mesh = plsc.VectorSubcoreMesh(core_axis_name="c", subcore_axis_name="s")

    @functools.partial(
        pl.kernel,
        mesh=mesh,
        out_type=jax.ShapeDtypeStruct((Q,), labels.dtype),
        scratch_types=[
            pltpu.VMEM((K,), labels.dtype),
            pltpu.VMEM((Q,), jnp.int32),
            pltpu.VMEM((Q,), labels.dtype),
        ],
        compiler_params=_sc_compiler_params(),
    )
    def gather_kernel(labels_hbm, idx_hbm, out_hbm, lab_v, idx_v, out_v):
        cid = lax.axis_index("c")
        sid = lax.axis_index("s")

        @pl.when(jnp.logical_and(cid == 0, sid == 0))
        def _():
            pltpu.sync_copy(labels_hbm, lab_v)
            pltpu.sync_copy(idx_hbm, idx_v)
            for j in range(Q // 16):
                ids = idx_v[pl.ds(j * 16, 16)]
                out_v[pl.ds(j * 16, 16)] = plsc.load_gather(lab_v, [ids])
            pltpu.sync_copy(out_v, out_hbm)

    return gather_kernel(labels, nn_idx)


def kernel(x, keys, labels):
    xsq = jnp.sum(x * x, axis=1, keepdims=True)
    nn_idx = _nn_argmin((x + x).T, xsq.T, keys.T)
    return _label_gather(labels, nn_idx.reshape(Q))


# speedup vs baseline: 1.8485x; 1.2895x over previous
import dataclasses
import functools

import jax
import jax.numpy as jnp
from jax import lax
from jax.experimental import pallas as pl
from jax.experimental.pallas import tpu as pltpu
from jax.experimental.pallas import tpu_sc as plsc

Q = 1024
D = 64
K = 100000
KB = 2048
NB = pl.cdiv(K, KB)
KTAIL = K - (NB - 1) * KB
SB = 8
IBIG = 2**30
FBIG = 3.0e38


CH = 2048
NCH = KB // CH
SPC = CH // SB


def _nn_body(x2t_ref, xsq_ref, keyst_ref, out_ref, rmin_ref, ridx_ref,
             m0_ref, k0_ref, m1_ref, k1_ref):
    i = pl.program_id(0)
    xsq = xsq_ref[...]

    @pl.when(i == 0)
    def _():
        rmin_ref[...] = jnp.full((SB, Q), FBIG, jnp.float32)
        ridx_ref[...] = jnp.zeros((SB, Q), jnp.int32)

    def compute_chunk(mw_ref, kw_ref, c):
        csl = slice(c * CH, (c + 1) * CH)
        kb = keyst_ref[:, csl].T
        kw_ref[csl] = jnp.sum(kb * kb, axis=1, keepdims=True)
        mw_ref[csl] = lax.dot_general(
            kb, x2t_ref[...],
            dimension_numbers=(((1,), (0,)), ((), ())),
            preferred_element_type=jnp.float32,
        )

    def step_mid(mw_ref, kw_ref, mr_ref, kr_ref):
        base = (i - 1) * KB
        acc = rmin_ref[...]
        aidx = ridx_ref[...]
        for c in range(NCH):
            compute_chunk(mw_ref, kw_ref, c)
            for s in range(c * SPC, (c + 1) * SPC):
                sl = slice(s * SB, (s + 1) * SB)
                d = (kr_ref[sl] + xsq) - mr_ref[sl]
                mask = d < acc
                acc = jnp.where(mask, d, acc)
                aidx = jnp.where(mask, base + s * SB, aidx)
        rmin_ref[...] = acc
        ridx_ref[...] = aidx

    @pl.when(i == 0)
    def _():
        for c in range(NCH):
            compute_chunk(m0_ref, k0_ref, c)

    @pl.when(jnp.logical_and(i > 0, jnp.logical_and(i < NB, i % 2 == 1)))
    def _():
        step_mid(m1_ref, k1_ref, m0_ref, k0_ref)

    @pl.when(jnp.logical_and(i > 0, jnp.logical_and(i < NB, i % 2 == 0)))
    def _():
        step_mid(m0_ref, k0_ref, m1_ref, k1_ref)

    @pl.when(i == NB)
    def _():
        mr_ref, kr_ref = (m0_ref, k0_ref) if (NB - 1) % 2 == 0 else (m1_ref, k1_ref)
        base = (NB - 1) * KB
        acc = rmin_ref[...]
        aidx = ridx_ref[...]
        for s in range(KTAIL // SB):
            sl = slice(s * SB, (s + 1) * SB)
            d = (kr_ref[sl] + xsq) - mr_ref[sl]
            mask = d < acc
            acc = jnp.where(mask, d, acc)
            aidx = jnp.where(mask, base + s * SB, aidx)
        gmin = jnp.min(acc, axis=0, keepdims=True)
        gidx = aidx + lax.broadcasted_iota(jnp.int32, (SB, Q), 0)
        cand = jnp.where(acc == gmin, gidx, IBIG)
        out_ref[...] = jnp.min(cand, axis=0, keepdims=True)


def _nn_argmin(x2t, xsq, keyst, interpret=False):
    return pl.pallas_call(
        _nn_body,
        grid=(NB + 1,),
        in_specs=[
            pl.BlockSpec((D, Q), lambda i: (0, 0)),
            pl.BlockSpec((1, Q), lambda i: (0, 0)),
            pl.BlockSpec((D, KB), lambda i: (0, jnp.minimum(i, NB - 1))),
        ],
        out_specs=pl.BlockSpec((1, Q), lambda i: (0, 0)),
        out_shape=jax.ShapeDtypeStruct((1, Q), jnp.int32),
        scratch_shapes=[
            pltpu.VMEM((SB, Q), jnp.float32),
            pltpu.VMEM((SB, Q), jnp.int32),
            pltpu.VMEM((KB, Q), jnp.float32),
            pltpu.VMEM((KB, 1), jnp.float32),
            pltpu.VMEM((KB, Q), jnp.float32),
            pltpu.VMEM((KB, 1), jnp.float32),
        ],
        interpret=interpret,
    )(x2t, xsq, keyst)


def _sc_compiler_params():
    cp = pltpu.CompilerParams()
    if "needs_layout_passes" in pltpu.CompilerParams.__dataclass_fields__:
        cp = dataclasses.replace(cp, needs_layout_passes=False)
    return cp


def _label_gather(labels, nn_idx):
    mesh = plsc.VectorSubcoreMesh(core_axis_name="c", subcore_axis_name="s")

    @functools.partial(
        pl.kernel,
        mesh=mesh,
        out_type=jax.ShapeDtypeStruct((Q,), labels.dtype),
        scratch_types=[
            pltpu.VMEM((K,), labels.dtype),
            pltpu.VMEM((Q,), jnp.int32),
            pltpu.VMEM((Q,), labels.dtype),
        ],
        compiler_params=_sc_compiler_params(),
    )
    def gather_kernel(labels_hbm, idx_hbm, out_hbm, lab_v, idx_v, out_v):
        cid = lax.axis_index("c")
        sid = lax.axis_index("s")

        @pl.when(jnp.logical_and(cid == 0, sid == 0))
        def _():
            pltpu.sync_copy(labels_hbm, lab_v)
            pltpu.sync_copy(idx_hbm, idx_v)
            for j in range(Q // 16):
                ids = idx_v[pl.ds(j * 16, 16)]
                out_v[pl.ds(j * 16, 16)] = plsc.load_gather(lab_v, [ids])
            pltpu.sync_copy(out_v, out_hbm)

    return gather_kernel(labels, nn_idx)


def kernel(x, keys, labels):
    xsq = jnp.sum(x * x, axis=1, keepdims=True)
    nn_idx = _nn_argmin((x + x).T, xsq.T, keys.T)
    return _label_gather(labels, nn_idx.reshape(Q))


# speedup vs baseline: 1.8567x; 1.0044x over previous
import dataclasses
import functools

import jax
import jax.numpy as jnp
from jax import lax
from jax.experimental import pallas as pl
from jax.experimental.pallas import tpu as pltpu
from jax.experimental.pallas import tpu_sc as plsc

Q = 1024
D = 64
K = 100000
KB = 4096
NB = pl.cdiv(K, KB)
KTAIL = K - (NB - 1) * KB
SB = 8
IBIG = 2**30
FBIG = 3.0e38


CH = 4096
NCH = KB // CH
SPC = CH // SB


def _nn_body(x2t_ref, xsq_ref, keyst_ref, out_ref, rmin_ref, ridx_ref,
             m0_ref, k0_ref, m1_ref, k1_ref):
    i = pl.program_id(0)
    xsq = xsq_ref[...]

    @pl.when(i == 0)
    def _():
        rmin_ref[...] = jnp.full((SB, Q), FBIG, jnp.float32)
        ridx_ref[...] = jnp.zeros((SB, Q), jnp.int32)

    def compute_chunk(mw_ref, kw_ref, c):
        csl = slice(c * CH, (c + 1) * CH)
        kb = keyst_ref[:, csl].T
        kw_ref[csl] = jnp.sum(kb * kb, axis=1, keepdims=True)
        mw_ref[csl] = lax.dot_general(
            kb, x2t_ref[...],
            dimension_numbers=(((1,), (0,)), ((), ())),
            preferred_element_type=jnp.float32,
        )

    def step_mid(mw_ref, kw_ref, mr_ref, kr_ref):
        base = (i - 1) * KB
        acc = rmin_ref[...]
        aidx = ridx_ref[...]
        for c in range(NCH):
            compute_chunk(mw_ref, kw_ref, c)
            for s in range(c * SPC, (c + 1) * SPC):
                sl = slice(s * SB, (s + 1) * SB)
                d = (kr_ref[sl] + xsq) - mr_ref[sl]
                mask = d < acc
                acc = jnp.where(mask, d, acc)
                aidx = jnp.where(mask, base + s * SB, aidx)
        rmin_ref[...] = acc
        ridx_ref[...] = aidx

    @pl.when(i == 0)
    def _():
        for c in range(NCH):
            compute_chunk(m0_ref, k0_ref, c)

    @pl.when(jnp.logical_and(i > 0, jnp.logical_and(i < NB, i % 2 == 1)))
    def _():
        step_mid(m1_ref, k1_ref, m0_ref, k0_ref)

    @pl.when(jnp.logical_and(i > 0, jnp.logical_and(i < NB, i % 2 == 0)))
    def _():
        step_mid(m0_ref, k0_ref, m1_ref, k1_ref)

    @pl.when(i == NB)
    def _():
        mr_ref, kr_ref = (m0_ref, k0_ref) if (NB - 1) % 2 == 0 else (m1_ref, k1_ref)
        base = (NB - 1) * KB
        acc = rmin_ref[...]
        aidx = ridx_ref[...]
        for s in range(KTAIL // SB):
            sl = slice(s * SB, (s + 1) * SB)
            d = (kr_ref[sl] + xsq) - mr_ref[sl]
            mask = d < acc
            acc = jnp.where(mask, d, acc)
            aidx = jnp.where(mask, base + s * SB, aidx)
        gmin = jnp.min(acc, axis=0, keepdims=True)
        gidx = aidx + lax.broadcasted_iota(jnp.int32, (SB, Q), 0)
        cand = jnp.where(acc == gmin, gidx, IBIG)
        out_ref[...] = jnp.min(cand, axis=0, keepdims=True)


def _nn_argmin(x2t, xsq, keyst, interpret=False):
    return pl.pallas_call(
        _nn_body,
        grid=(NB + 1,),
        in_specs=[
            pl.BlockSpec((D, Q), lambda i: (0, 0)),
            pl.BlockSpec((1, Q), lambda i: (0, 0)),
            pl.BlockSpec((D, KB), lambda i: (0, jnp.minimum(i, NB - 1))),
        ],
        out_specs=pl.BlockSpec((1, Q), lambda i: (0, 0)),
        out_shape=jax.ShapeDtypeStruct((1, Q), jnp.int32),
        scratch_shapes=[
            pltpu.VMEM((SB, Q), jnp.float32),
            pltpu.VMEM((SB, Q), jnp.int32),
            pltpu.VMEM((KB, Q), jnp.float32),
            pltpu.VMEM((KB, 1), jnp.float32),
            pltpu.VMEM((KB, Q), jnp.float32),
            pltpu.VMEM((KB, 1), jnp.float32),
        ],
        interpret=interpret,
    )(x2t, xsq, keyst)


def _sc_compiler_params():
    cp = pltpu.CompilerParams()
    if "needs_layout_passes" in pltpu.CompilerParams.__dataclass_fields__:
        cp = dataclasses.replace(cp, needs_layout_passes=False)
    return cp


def _label_gather(labels, nn_idx):
    mesh = plsc.VectorSubcoreMesh(core_axis_name="c", subcore_axis_name="s")

    @functools.partial(
        pl.kernel,
        mesh=mesh,
        out_type=jax.ShapeDtypeStruct((Q,), labels.dtype),
        scratch_types=[
            pltpu.VMEM((K,), labels.dtype),
            pltpu.VMEM((Q,), jnp.int32),
            pltpu.VMEM((Q,), labels.dtype),
        ],
        compiler_params=_sc_compiler_params(),
    )
    def gather_kernel(labels_hbm, idx_hbm, out_hbm, lab_v, idx_v, out_v):
        cid = lax.axis_index("c")
        sid = lax.axis_index("s")

        @pl.when(jnp.logical_and(cid == 0, sid == 0))
        def _():
            pltpu.sync_copy(labels_hbm, lab_v)
            pltpu.sync_copy(idx_hbm, idx_v)
            for j in range(Q // 16):
                ids = idx_v[pl.ds(j * 16, 16)]
                out_v[pl.ds(j * 16, 16)] = plsc.load_gather(lab_v, [ids])
            pltpu.sync_copy(out_v, out_hbm)

    return gather_kernel(labels, nn_idx)


def kernel(x, keys, labels):
    xsq = jnp.sum(x * x, axis=1, keepdims=True)
    nn_idx = _nn_argmin((x + x).T, xsq.T, keys.T)
    return _label_gather(labels, nn_idx.reshape(Q))
